# baseline (speedup 1.0000x reference)
import functools
import math

import jax
import jax.numpy as jnp
from jax import lax
from jax.experimental import pallas as pl
from jax.experimental.pallas import tpu as pltpu

_B, _S, _EMB, _H, _INTER = 64, 576, 1152, 16, 4304
_HD = _EMB // _H
_F32 = jnp.float32


def _prep_body(probe_ref, wq_ref, bq_ref, wk_ref, wsct_ref):
    q_row = lax.dot_general(
        probe_ref[...], wq_ref[...], (((1,), (1,)), ((), ())),
        preferred_element_type=_F32,
    ) + bq_ref[...]
    head = lax.broadcasted_iota(jnp.int32, (_H, _EMB), 0)
    dim = lax.broadcasted_iota(jnp.int32, (_H, _EMB), 1)
    qmat = jnp.where(dim // _HD == head, 1.0, 0.0).astype(_F32) * q_row
    wsct = lax.dot_general(
        qmat, wk_ref[...], (((1,), (0,)), ((), ())),
        preferred_element_type=_F32,
    )
    wsct_ref[...] = wsct * _F32(1.0 / math.sqrt(_HD))


_NBUF = 6


def _pool_body(hid_hbm, wsct_ref, pooled_ref, buf_ref, sem):
    b = pl.program_id(0)

    @pl.when(b == 0)
    def _():
        for k in range(_NBUF):
            pltpu.make_async_copy(
                hid_hbm.at[k], buf_ref.at[k], sem.at[k],
            ).start()

    slot = lax.rem(b, _NBUF)
    pltpu.make_async_copy(hid_hbm.at[b], buf_ref.at[slot], sem.at[slot]).wait()

    hsb = buf_ref[slot].astype(jnp.bfloat16)
    wb = wsct_ref[...].astype(jnp.bfloat16)
    scores = lax.dot_general(
        hsb, wb, (((1,), (1,)), ((), ())),
        preferred_element_type=_F32,
    )
    m = jnp.max(scores, axis=0, keepdims=True)
    p = jnp.exp(scores - m)
    a = p / jnp.sum(p, axis=0, keepdims=True)
    pooled = lax.dot_general(
        a.astype(jnp.bfloat16), hsb, (((0,), (0,)), ((), ())),
        preferred_element_type=_F32,
    )
    pooled_ref[0] = pooled

    nxt = b + _NBUF

    @pl.when(nxt < _B)
    def _():
        pltpu.make_async_copy(
            hid_hbm.at[nxt], buf_ref.at[slot], sem.at[slot],
        ).start()


_FCH = _INTER // 2


def _tail_body(pooled_hbm, wv_hbm, bv_ref, outw_hbm, outb_ref, g_ref, b_ref,
               fc1_hbm, fc1b_ref, fc2_hbm, fc2b_ref, out_ref,
               pooled_v, wv_v, outw_v, fc1_v, fc2_v, sems):
    cps = [
        pltpu.make_async_copy(pooled_hbm, pooled_v, sems.at[0]),
        pltpu.make_async_copy(wv_hbm, wv_v, sems.at[1]),
        pltpu.make_async_copy(outw_hbm, outw_v, sems.at[2]),
        pltpu.make_async_copy(fc1_hbm.at[pl.ds(0, _FCH)],
                              fc1_v.at[pl.ds(0, _FCH)], sems.at[3]),
        pltpu.make_async_copy(fc1_hbm.at[pl.ds(_FCH, _FCH)],
                              fc1_v.at[pl.ds(_FCH, _FCH)], sems.at[4]),
        pltpu.make_async_copy(fc2_hbm, fc2_v, sems.at[5]),
    ]
    for cp in cps:
        cp.start()

    cps[0].wait()
    cps[1].wait()
    parts = []
    for i in range(_H):
        p_i = pooled_v[:, i, :]
        w_i = wv_v[i * _HD:(i + 1) * _HD, :]
        parts.append(lax.dot_general(
            p_i, w_i, (((1,), (1,)), ((), ())), preferred_element_type=_F32,
        ))
    o = jnp.concatenate(parts, axis=1) + bv_ref[...]
    cps[2].wait()
    o = lax.dot_general(
        o, outw_v[...], (((1,), (1,)), ((), ())),
        preferred_element_type=_F32,
    ) + outb_ref[...]
    residual = o
    mu = jnp.mean(o, axis=1, keepdims=True)
    xc = o - mu
    var = jnp.mean(xc * xc, axis=1, keepdims=True)
    hn = xc * lax.rsqrt(var + 1e-5) * g_ref[...] + b_ref[...]
    cps[3].wait()
    cps[4].wait()
    h1 = lax.dot_general(
        hn, fc1_v[...], (((1,), (1,)), ((), ())),
        preferred_element_type=_F32,
    ) + fc1b_ref[...]
    h1 = jax.nn.gelu(h1, approximate=True)
    cps[5].wait()
    m = lax.dot_general(
        h1, fc2_v[...], (((1,), (1,)), ((), ())),
        preferred_element_type=_F32,
    ) + fc2b_ref[...]
    out_ref[...] = residual + m


@jax.jit
def kernel(hidden_state, probe, in_proj_w, in_proj_b, out_proj_w, out_proj_b,
           ln_g, ln_b, fc1_w, fc1_b, fc2_w, fc2_b):
    e = _EMB
    wq, wk, wv = in_proj_w[:e], in_proj_w[e:2 * e], in_proj_w[2 * e:]
    bq = in_proj_b[:e].reshape(1, e)
    bv = in_proj_b[2 * e:].reshape(1, e)
    probe_row = probe.reshape(1, e)

    wsct = pl.pallas_call(
        _prep_body,
        out_shape=jax.ShapeDtypeStruct((_H, _EMB), _F32),
    )(probe_row, wq, bq, wk)

    pooled = pl.pallas_call(
        _pool_body,
        grid=(_B,),
        in_specs=[
            pl.BlockSpec(memory_space=pltpu.MemorySpace.HBM),
            pl.BlockSpec((_H, _EMB), lambda b: (0, 0)),
        ],
        out_specs=pl.BlockSpec((1, _H, _EMB), lambda b: (b, 0, 0)),
        out_shape=jax.ShapeDtypeStruct((_B, _H, _EMB), _F32),
        scratch_shapes=[
            pltpu.VMEM((_NBUF, _S, _EMB), _F32),
            pltpu.SemaphoreType.DMA((_NBUF,)),
        ],
    )(hidden_state, wsct)

    hbm = pl.BlockSpec(memory_space=pltpu.MemorySpace.HBM)
    vm = pl.BlockSpec(memory_space=pltpu.MemorySpace.VMEM)
    out = pl.pallas_call(
        _tail_body,
        in_specs=[hbm, hbm, vm, hbm, vm, vm, vm, hbm, vm, hbm, vm],
        out_shape=jax.ShapeDtypeStruct((_B, _EMB), _F32),
        scratch_shapes=[
            pltpu.VMEM((_B, _H, _EMB), _F32),
            pltpu.VMEM((_EMB, _EMB), _F32),
            pltpu.VMEM((_EMB, _EMB), _F32),
            pltpu.VMEM((_INTER, _EMB), _F32),
            pltpu.VMEM((_EMB, _INTER), _F32),
            pltpu.SemaphoreType.DMA((6,)),
        ],
        compiler_params=pltpu.CompilerParams(
            vmem_limit_bytes=100 * 1024 * 1024,
        ),
    )(pooled, wv, bv, out_proj_w, out_proj_b.reshape(1, e), ln_g.reshape(1, e),
      ln_b.reshape(1, e), fc1_w, fc1_b.reshape(1, _INTER), fc2_w,
      fc2_b.reshape(1, e))

    return out


# speedup vs baseline: 1.0730x; 1.0730x over previous
import functools
import math

import jax
import jax.numpy as jnp
from jax import lax
from jax.experimental import pallas as pl
from jax.experimental.pallas import tpu as pltpu

_B, _S, _EMB, _H, _INTER = 64, 576, 1152, 16, 4304
_HD = _EMB // _H
_F32 = jnp.float32


def _prep_body(probe_ref, wq_ref, bq_ref, wk_ref, wsct_ref):
    q_row = lax.dot_general(
        probe_ref[...], wq_ref[...], (((1,), (1,)), ((), ())),
        preferred_element_type=_F32,
    ) + bq_ref[...]
    head = lax.broadcasted_iota(jnp.int32, (_H, _EMB), 0)
    dim = lax.broadcasted_iota(jnp.int32, (_H, _EMB), 1)
    qmat = jnp.where(dim // _HD == head, 1.0, 0.0).astype(_F32) * q_row
    wsct = lax.dot_general(
        qmat, wk_ref[...], (((1,), (0,)), ((), ())),
        preferred_element_type=_F32,
    )
    wsct_ref[...] = wsct * _F32(1.0 / math.sqrt(_HD))


_NBUF = 6


def _pool_body(hid_hbm, wsct_ref, pooled_ref, buf_ref, sem):
    b = pl.program_id(0)

    @pl.when(b == 0)
    def _():
        for k in range(_NBUF):
            pltpu.make_async_copy(
                hid_hbm.at[k], buf_ref.at[k], sem.at[k],
            ).start()

    slot = lax.rem(b, _NBUF)
    pltpu.make_async_copy(hid_hbm.at[b], buf_ref.at[slot], sem.at[slot]).wait()

    hsb = buf_ref[slot].astype(jnp.bfloat16)
    wb = wsct_ref[...].astype(jnp.bfloat16)
    scores = lax.dot_general(
        hsb, wb, (((1,), (1,)), ((), ())),
        preferred_element_type=_F32,
    )
    m = jnp.max(scores, axis=0, keepdims=True)
    p = jnp.exp(scores - m)
    a = p / jnp.sum(p, axis=0, keepdims=True)
    pooled = lax.dot_general(
        a.astype(jnp.bfloat16), hsb, (((0,), (0,)), ((), ())),
        preferred_element_type=_F32,
    )
    pooled_ref[0] = pooled

    nxt = b + _NBUF

    @pl.when(nxt < _B)
    def _():
        pltpu.make_async_copy(
            hid_hbm.at[nxt], buf_ref.at[slot], sem.at[slot],
        ).start()


_FCH = _INTER // 2


def _tail_body(pooled_hbm, wv_hbm, bv_ref, outw_hbm, outb_ref, g_ref, b_ref,
               fc1_hbm, fc1b_ref, fc2_hbm, fc2b_ref, out_ref,
               pooled_v, wv_v, outw_v, fc1_v, fc2_v, sems):
    cps = [
        pltpu.make_async_copy(pooled_hbm, pooled_v, sems.at[0]),
        pltpu.make_async_copy(wv_hbm.at[pl.ds(2 * _EMB, _EMB)], wv_v,
                              sems.at[1]),
        pltpu.make_async_copy(outw_hbm, outw_v, sems.at[2]),
        pltpu.make_async_copy(fc1_hbm.at[pl.ds(0, _FCH)],
                              fc1_v.at[pl.ds(0, _FCH)], sems.at[3]),
        pltpu.make_async_copy(fc1_hbm.at[pl.ds(_FCH, _FCH)],
                              fc1_v.at[pl.ds(_FCH, _FCH)], sems.at[4]),
        pltpu.make_async_copy(fc2_hbm, fc2_v, sems.at[5]),
    ]
    for cp in cps:
        cp.start()

    cps[0].wait()
    cps[1].wait()
    parts = []
    for i in range(_H):
        p_i = pooled_v[:, i, :]
        w_i = wv_v[i * _HD:(i + 1) * _HD, :]
        parts.append(lax.dot_general(
            p_i, w_i, (((1,), (1,)), ((), ())), preferred_element_type=_F32,
        ))
    o = jnp.concatenate(parts, axis=1) + bv_ref[...]
    cps[2].wait()
    o = lax.dot_general(
        o, outw_v[...], (((1,), (1,)), ((), ())),
        preferred_element_type=_F32,
    ) + outb_ref[...]
    residual = o
    mu = jnp.mean(o, axis=1, keepdims=True)
    xc = o - mu
    var = jnp.mean(xc * xc, axis=1, keepdims=True)
    hn = xc * lax.rsqrt(var + 1e-5) * g_ref[...] + b_ref[...]
    cps[3].wait()
    cps[4].wait()
    h1 = lax.dot_general(
        hn, fc1_v[...], (((1,), (1,)), ((), ())),
        preferred_element_type=_F32,
    ) + fc1b_ref[...]
    h1 = jax.nn.gelu(h1, approximate=True)
    cps[5].wait()
    m = lax.dot_general(
        h1, fc2_v[...], (((1,), (1,)), ((), ())),
        preferred_element_type=_F32,
    ) + fc2b_ref[...]
    out_ref[...] = residual + m


@jax.jit
def kernel(hidden_state, probe, in_proj_w, in_proj_b, out_proj_w, out_proj_b,
           ln_g, ln_b, fc1_w, fc1_b, fc2_w, fc2_b):
    e = _EMB
    bq = in_proj_b[:e].reshape(1, e)
    bv = in_proj_b[2 * e:].reshape(1, e)
    probe_row = probe.reshape(1, e)

    wsct = pl.pallas_call(
        _prep_body,
        grid=(1,),
        in_specs=[
            pl.BlockSpec((1, _EMB), lambda i: (0, 0)),
            pl.BlockSpec((_EMB, _EMB), lambda i: (0, 0)),
            pl.BlockSpec((1, _EMB), lambda i: (0, 0)),
            pl.BlockSpec((_EMB, _EMB), lambda i: (1, 0)),
        ],
        out_specs=pl.BlockSpec((_H, _EMB), lambda i: (0, 0)),
        out_shape=jax.ShapeDtypeStruct((_H, _EMB), _F32),
    )(probe_row, in_proj_w, bq, in_proj_w)

    pooled = pl.pallas_call(
        _pool_body,
        grid=(_B,),
        in_specs=[
            pl.BlockSpec(memory_space=pltpu.MemorySpace.HBM),
            pl.BlockSpec((_H, _EMB), lambda b: (0, 0)),
        ],
        out_specs=pl.BlockSpec((1, _H, _EMB), lambda b: (b, 0, 0)),
        out_shape=jax.ShapeDtypeStruct((_B, _H, _EMB), _F32),
        scratch_shapes=[
            pltpu.VMEM((_NBUF, _S, _EMB), _F32),
            pltpu.SemaphoreType.DMA((_NBUF,)),
        ],
    )(hidden_state, wsct)

    hbm = pl.BlockSpec(memory_space=pltpu.MemorySpace.HBM)
    vm = pl.BlockSpec(memory_space=pltpu.MemorySpace.VMEM)
    out = pl.pallas_call(
        _tail_body,
        in_specs=[hbm, hbm, vm, hbm, vm, vm, vm, hbm, vm, hbm, vm],
        out_shape=jax.ShapeDtypeStruct((_B, _EMB), _F32),
        scratch_shapes=[
            pltpu.VMEM((_B, _H, _EMB), _F32),
            pltpu.VMEM((_EMB, _EMB), _F32),
            pltpu.VMEM((_EMB, _EMB), _F32),
            pltpu.VMEM((_INTER, _EMB), _F32),
            pltpu.VMEM((_EMB, _INTER), _F32),
            pltpu.SemaphoreType.DMA((6,)),
        ],
        compiler_params=pltpu.CompilerParams(
            vmem_limit_bytes=100 * 1024 * 1024,
        ),
    )(pooled, in_proj_w, bv, out_proj_w, out_proj_b.reshape(1, e), ln_g.reshape(1, e),
      ln_b.reshape(1, e), fc1_w, fc1_b.reshape(1, _INTER), fc2_w,
      fc2_b.reshape(1, e))

    return out


# speedup vs baseline: 1.0779x; 1.0046x over previous
import math

import jax
import jax.numpy as jnp
from jax import lax
from jax.experimental import pallas as pl
from jax.experimental.pallas import tpu as pltpu

_B, _S, _EMB, _H, _INTER = 64, 576, 1152, 16, 4304
_HD = _EMB // _H
_F32 = jnp.float32
_NBUF = 3
_FCH = _INTER // 2


def _wv_copy(ipw_hbm, wv_v, wsem):
    return pltpu.make_async_copy(
        ipw_hbm.at[pl.ds(2 * _EMB, _EMB)], wv_v, wsem.at[1])


def _fc1_copies(fc1_hbm, fc1_v, wsem):
    return (
        pltpu.make_async_copy(fc1_hbm.at[pl.ds(0, _FCH)],
                              fc1_v.at[pl.ds(0, _FCH)], wsem.at[3]),
        pltpu.make_async_copy(fc1_hbm.at[pl.ds(_FCH, _FCH)],
                              fc1_v.at[pl.ds(_FCH, _FCH)], wsem.at[4]),
    )


def _fused_body(probe_ref, bq_ref, bv_ref, outb_ref, g_ref, b2_ref,
                fc1b_ref, fc2b_ref,
                hid_hbm, ipw_hbm, outw_hbm, fc1_hbm, fc2_hbm,
                out_ref,
                buf_ref, pooled_v, wsct_v, wv_v, outw_v, fc1_v, fc2_v,
                hsem, wsem):
    b = pl.program_id(0)

    @pl.when(b == 0)
    def _prologue():
        qk_cp = pltpu.make_async_copy(
            ipw_hbm.at[pl.ds(0, 2 * _EMB)],
            fc1_v.at[pl.ds(0, 2 * _EMB)], wsem.at[0])
        qk_cp.start()
        for k in range(_NBUF):
            pltpu.make_async_copy(
                hid_hbm.at[k], buf_ref.at[k], hsem.at[k]).start()
        _wv_copy(ipw_hbm, wv_v, wsem).start()
        pltpu.make_async_copy(outw_hbm, outw_v, wsem.at[2]).start()
        pltpu.make_async_copy(fc2_hbm, fc2_v, wsem.at[5]).start()
        qk_cp.wait()
        q_row = lax.dot_general(
            probe_ref[...], fc1_v[0:_EMB, :], (((1,), (1,)), ((), ())),
            preferred_element_type=_F32,
        ) + bq_ref[...]
        head = lax.broadcasted_iota(jnp.int32, (_H, _EMB), 0)
        dim = lax.broadcasted_iota(jnp.int32, (_H, _EMB), 1)
        qmat = jnp.where(dim // _HD == head, 1.0, 0.0).astype(_F32) * q_row
        wsct = lax.dot_general(
            qmat, fc1_v[_EMB:2 * _EMB, :], (((1,), (0,)), ((), ())),
            preferred_element_type=_F32,
        )
        wsct_v[...] = wsct * _F32(1.0 / math.sqrt(_HD))
        c1, c2 = _fc1_copies(fc1_hbm, fc1_v, wsem)
        c1.start()
        c2.start()

    slot = lax.rem(b, _NBUF)
    pltpu.make_async_copy(hid_hbm.at[b], buf_ref.at[slot], hsem.at[slot]).wait()

    hsb = buf_ref[slot].astype(jnp.bfloat16)
    wb = wsct_v[...].astype(jnp.bfloat16)
    scores = lax.dot_general(
        hsb, wb, (((1,), (1,)), ((), ())), preferred_element_type=_F32,
    )
    m = jnp.max(scores, axis=0, keepdims=True)
    p = jnp.exp(scores - m)
    a = p / jnp.sum(p, axis=0, keepdims=True)
    pooled = lax.dot_general(
        a.astype(jnp.bfloat16), hsb, (((0,), (0,)), ((), ())),
        preferred_element_type=_F32,
    )
    pooled_v[b] = pooled

    nxt = b + _NBUF

    @pl.when(nxt < _B)
    def _next_copy():
        pltpu.make_async_copy(
            hid_hbm.at[nxt], buf_ref.at[slot], hsem.at[slot]).start()

    @pl.when(b == _B - 1)
    def _tail():
        _wv_copy(ipw_hbm, wv_v, wsem).wait()
        parts = []
        for i in range(_H):
            p_i = pooled_v[:, i, :]
            w_i = wv_v[i * _HD:(i + 1) * _HD, :]
            parts.append(lax.dot_general(
                p_i, w_i, (((1,), (1,)), ((), ())),
                preferred_element_type=_F32,
            ))
        o = jnp.concatenate(parts, axis=1) + bv_ref[...]
        pltpu.make_async_copy(outw_hbm, outw_v, wsem.at[2]).wait()
        o = lax.dot_general(
            o, outw_v[...], (((1,), (1,)), ((), ())),
            preferred_element_type=_F32,
        ) + outb_ref[...]
        residual = o
        mu = jnp.mean(o, axis=1, keepdims=True)
        xc = o - mu
        var = jnp.mean(xc * xc, axis=1, keepdims=True)
        hn = xc * lax.rsqrt(var + 1e-5) * g_ref[...] + b2_ref[...]
        c1, c2 = _fc1_copies(fc1_hbm, fc1_v, wsem)
        c1.wait()
        c2.wait()
        h1 = lax.dot_general(
            hn, fc1_v[...], (((1,), (1,)), ((), ())),
            preferred_element_type=_F32,
        ) + fc1b_ref[...]
        h1 = jax.nn.gelu(h1, approximate=True)
        pltpu.make_async_copy(fc2_hbm, fc2_v, wsem.at[5]).wait()
        mlp = lax.dot_general(
            h1, fc2_v[...], (((1,), (1,)), ((), ())),
            preferred_element_type=_F32,
        ) + fc2b_ref[...]
        out_ref[...] = residual + mlp


@jax.jit
def kernel(hidden_state, probe, in_proj_w, in_proj_b, out_proj_w, out_proj_b,
           ln_g, ln_b, fc1_w, fc1_b, fc2_w, fc2_b):
    e = _EMB
    row = lambda x: x.reshape(1, -1)
    vrow = pl.BlockSpec((1, e), lambda b: (0, 0))
    hbm = pl.BlockSpec(memory_space=pltpu.MemorySpace.HBM)
    out = pl.pallas_call(
        _fused_body,
        grid=(_B,),
        in_specs=[vrow, vrow, vrow, vrow, vrow, vrow,
                  pl.BlockSpec((1, _INTER), lambda b: (0, 0)), vrow,
                  hbm, hbm, hbm, hbm, hbm],
        out_specs=pl.BlockSpec((_B, _EMB), lambda b: (0, 0)),
        out_shape=jax.ShapeDtypeStruct((_B, _EMB), _F32),
        scratch_shapes=[
            pltpu.VMEM((_NBUF, _S, _EMB), _F32),
            pltpu.VMEM((_B, _H, _EMB), _F32),
            pltpu.VMEM((_H, _EMB), _F32),
            pltpu.VMEM((_EMB, _EMB), _F32),
            pltpu.VMEM((_EMB, _EMB), _F32),
            pltpu.VMEM((_INTER, _EMB), _F32),
            pltpu.VMEM((_EMB, _INTER), _F32),
            pltpu.SemaphoreType.DMA((_NBUF,)),
            pltpu.SemaphoreType.DMA((6,)),
        ],
        compiler_params=pltpu.CompilerParams(
            vmem_limit_bytes=64 * 1024 * 1024,
        ),
    )(row(probe), row(in_proj_b[:e]), row(in_proj_b[2 * e:]),
      row(out_proj_b), row(ln_g), row(ln_b), row(fc1_b), row(fc2_b),
      hidden_state, in_proj_w, out_proj_w, fc1_w, fc2_w)
    return out
